# baseline (speedup 1.0000x reference)
import functools

import jax
import jax.numpy as jnp
from jax import lax
from jax.experimental import pallas as pl
from jax.experimental.pallas import tpu as pltpu
from jax.experimental.pallas import tpu_sc as plsc

VOCAB = 100000
HIDDEN = 32
BATCH = 1024


LANES = 16
N_CHUNK = BATCH // LANES


def _sc_gather_t(table_flat, ids):
    info = plsc.get_sparse_core_info()
    nc, ns = info.num_cores, info.num_subcores
    assert nc * ns == HIDDEN

    mesh = plsc.VectorSubcoreMesh(core_axis_name="c", subcore_axis_name="s")

    @functools.partial(
        pl.kernel,
        mesh=mesh,
        compiler_params=pltpu.CompilerParams(use_tc_tiling_on_sc=False),
        out_type=jax.ShapeDtypeStruct((HIDDEN, BATCH), jnp.float32),
        scratch_types=[
            pltpu.VMEM((BATCH,), jnp.int32),
            pltpu.VMEM((BATCH,), jnp.int32),
            pltpu.VMEM((BATCH,), jnp.float32),
            pltpu.SemaphoreType.DMA,
        ],
    )
    def gather_kernel(table_hbm, idx_hbm, out_hbm, ids_v, aidx_v, out_v, sem):
        h = lax.axis_index("s") * nc + lax.axis_index("c")
        pltpu.sync_copy(idx_hbm, ids_v)
        base = h * VOCAB
        for c in range(N_CHUNK):
            idv = ids_v[pl.ds(c * LANES, LANES)]
            aidx_v[pl.ds(c * LANES, LANES)] = base + idv
        pltpu.async_copy(table_hbm.at[aidx_v], out_v, sem).wait()
        pltpu.sync_copy(out_v, out_hbm.at[h])

    return gather_kernel(table_flat, ids)


V_TILE = 4096
NBUF = 2
GRID = pl.cdiv(VOCAB, V_TILE)
V_LAST = VOCAB - (GRID - 1) * V_TILE


def _matmul_body(embt_ref, wt_ref, out_ref, obuf, sems):
    i = pl.program_id(0)
    slot = lax.rem(i, NBUF)

    @pl.when(i >= NBUF)
    def _():
        pltpu.make_async_copy(
            obuf.at[slot],
            out_ref.at[pl.ds((i - NBUF) * V_TILE, V_TILE), :],
            sems.at[slot],
        ).wait()

    obuf[slot] = lax.dot_general(
        wt_ref[...],
        embt_ref[...],
        dimension_numbers=(((0,), (0,)), ((), ())),
        preferred_element_type=jnp.float32,
    )

    @pl.when(i < GRID - 1)
    def _():
        pltpu.make_async_copy(
            obuf.at[slot],
            out_ref.at[pl.ds(i * V_TILE, V_TILE), :],
            sems.at[slot],
        ).start()

    @pl.when(i == GRID - 1)
    def _():
        pltpu.make_async_copy(
            obuf.at[slot, :V_LAST],
            out_ref.at[pl.ds(i * V_TILE, V_LAST), :],
            sems.at[slot],
        ).start()
        for s in range(NBUF):
            step = GRID - 1 - ((GRID - 1 - s) % NBUF)
            if step == GRID - 1:
                pltpu.make_async_copy(
                    obuf.at[s, :V_LAST],
                    out_ref.at[pl.ds(step * V_TILE, V_LAST), :],
                    sems.at[s],
                ).wait()
            else:
                pltpu.make_async_copy(
                    obuf.at[s],
                    out_ref.at[pl.ds(step * V_TILE, V_TILE), :],
                    sems.at[s],
                ).wait()


def _tc_matmul_t(Wt, embt):
    return pl.pallas_call(
        _matmul_body,
        grid=(GRID,),
        in_specs=[
            pl.BlockSpec((HIDDEN, BATCH), lambda i: (0, 0)),
            pl.BlockSpec((HIDDEN, V_TILE), lambda i: (0, i)),
        ],
        out_specs=pl.BlockSpec(memory_space=pltpu.MemorySpace.HBM),
        out_shape=jax.ShapeDtypeStruct((VOCAB, BATCH), jnp.float32),
        scratch_shapes=[
            pltpu.VMEM((NBUF, V_TILE, BATCH), jnp.float32),
            pltpu.SemaphoreType.DMA((NBUF,)),
        ],
    )(embt, Wt)


def kernel(input_ids, W):
    ids = input_ids.astype(jnp.int32)
    Wt = W.T
    table_flat = Wt.reshape(HIDDEN * VOCAB)
    embt = _sc_gather_t(table_flat, ids)
    logits_t = _tc_matmul_t(Wt, embt)
    return logits_t.T


# speedup vs baseline: 1.0105x; 1.0105x over previous
import functools

import jax
import jax.numpy as jnp
from jax import lax
from jax.experimental import pallas as pl
from jax.experimental.pallas import tpu as pltpu
from jax.experimental.pallas import tpu_sc as plsc

VOCAB = 100000
HIDDEN = 32
BATCH = 1024


LANES = 16
N_CHUNK = BATCH // LANES


def _sc_gather_t(table_flat, ids):
    info = plsc.get_sparse_core_info()
    nc, ns = info.num_cores, info.num_subcores
    assert nc * ns == HIDDEN

    mesh = plsc.VectorSubcoreMesh(core_axis_name="c", subcore_axis_name="s")

    @functools.partial(
        pl.kernel,
        mesh=mesh,
        compiler_params=pltpu.CompilerParams(use_tc_tiling_on_sc=False),
        out_type=jax.ShapeDtypeStruct((HIDDEN, BATCH), jnp.float32),
        scratch_types=[
            pltpu.VMEM((BATCH,), jnp.int32),
            pltpu.VMEM((BATCH,), jnp.int32),
            pltpu.VMEM((BATCH,), jnp.float32),
            pltpu.SemaphoreType.DMA,
        ],
    )
    def gather_kernel(table_hbm, idx_hbm, out_hbm, ids_v, aidx_v, out_v, sem):
        h = lax.axis_index("s") * nc + lax.axis_index("c")
        pltpu.sync_copy(idx_hbm, ids_v)
        base = h * VOCAB
        for c in range(N_CHUNK):
            idv = ids_v[pl.ds(c * LANES, LANES)]
            aidx_v[pl.ds(c * LANES, LANES)] = base + idv
        pltpu.async_copy(table_hbm.at[aidx_v], out_v, sem).wait()
        pltpu.sync_copy(out_v, out_hbm.at[h])

    return gather_kernel(table_flat, ids)


V_TILE = 2048
NBUF = 4
GRID = pl.cdiv(VOCAB, V_TILE)
V_LAST = VOCAB - (GRID - 1) * V_TILE


def _matmul_body(embt_ref, wt_ref, out_ref, obuf, sems):
    i = pl.program_id(0)
    slot = lax.rem(i, NBUF)

    @pl.when(i >= NBUF)
    def _():
        pltpu.make_async_copy(
            obuf.at[slot],
            out_ref.at[pl.ds((i - NBUF) * V_TILE, V_TILE), :],
            sems.at[slot],
        ).wait()

    obuf[slot] = lax.dot_general(
        wt_ref[...],
        embt_ref[...],
        dimension_numbers=(((0,), (0,)), ((), ())),
        preferred_element_type=jnp.float32,
    )

    @pl.when(i < GRID - 1)
    def _():
        pltpu.make_async_copy(
            obuf.at[slot],
            out_ref.at[pl.ds(i * V_TILE, V_TILE), :],
            sems.at[slot],
        ).start()

    @pl.when(i == GRID - 1)
    def _():
        pltpu.make_async_copy(
            obuf.at[slot, :V_LAST],
            out_ref.at[pl.ds(i * V_TILE, V_LAST), :],
            sems.at[slot],
        ).start()
        for s in range(NBUF):
            step = GRID - 1 - ((GRID - 1 - s) % NBUF)
            if step == GRID - 1:
                pltpu.make_async_copy(
                    obuf.at[s, :V_LAST],
                    out_ref.at[pl.ds(step * V_TILE, V_LAST), :],
                    sems.at[s],
                ).wait()
            else:
                pltpu.make_async_copy(
                    obuf.at[s],
                    out_ref.at[pl.ds(step * V_TILE, V_TILE), :],
                    sems.at[s],
                ).wait()


def _tc_matmul_t(Wt, embt):
    return pl.pallas_call(
        _matmul_body,
        grid=(GRID,),
        in_specs=[
            pl.BlockSpec((HIDDEN, BATCH), lambda i: (0, 0)),
            pl.BlockSpec((HIDDEN, V_TILE), lambda i: (0, i)),
        ],
        out_specs=pl.BlockSpec(memory_space=pltpu.MemorySpace.HBM),
        out_shape=jax.ShapeDtypeStruct((VOCAB, BATCH), jnp.float32),
        scratch_shapes=[
            pltpu.VMEM((NBUF, V_TILE, BATCH), jnp.float32),
            pltpu.SemaphoreType.DMA((NBUF,)),
        ],
    )(embt, Wt)


def kernel(input_ids, W):
    ids = input_ids.astype(jnp.int32)
    Wt = W.T
    table_flat = Wt.reshape(HIDDEN * VOCAB)
    embt = _sc_gather_t(table_flat, ids)
    logits_t = _tc_matmul_t(Wt, embt)
    return logits_t.T
